# Initial kernel scaffold; baseline (speedup 1.0000x reference)
import jax
import jax.numpy as jnp
from jax.experimental import pallas as pl


def kernel(x, edge_index, params):
    raise NotImplementedError("write your pallas kernel here")


# speedup vs baseline: 1.9912x; 1.9912x over previous
import functools

import jax
import jax.numpy as jnp
from jax import lax
from jax.experimental import pallas as pl
from jax.experimental.pallas import tpu as pltpu
from jax.experimental.pallas import tpu_sc as plsc

_NC = 2
_NS = 16
_NW = _NC * _NS
_LANES = 16

_H = 512
_HB = _H // _LANES
_NPW = 320
_OFF_STAGE = 336
_SRCS_BUF = 12816


_GATHER_DNUMS = lax.GatherDimensionNumbers(
    offset_dims=(), collapsed_slice_dims=(0,), start_index_map=(0,))


def _shuf(v, idx):
    return lax.gather(v, idx[:, None], _GATHER_DNUMS, slice_sizes=(1,),
                      mode=lax.GatherScatterMode.PROMISE_IN_BOUNDS)


def _bfly(v, op):
    idx = lax.iota(jnp.int32, _LANES)
    for sh in (8, 4, 2, 1):
        v = op(v, _shuf(v, idx ^ sh))
    return v


def _splat(v, j):
    return _shuf(v, jnp.full((_LANES,), j, jnp.int32))


def _gat_sc(xl, xr, srcs_p, offs_p, att, bias):
    n = xr.shape[0]

    mesh = plsc.VectorSubcoreMesh(core_axis_name="c", subcore_axis_name="s")

    @functools.partial(
        pl.kernel,
        mesh=mesh,
        out_type=jax.ShapeDtypeStruct((n, _H), jnp.float32),
        scratch_types=[
            pltpu.VMEM((_OFF_STAGE,), jnp.int32),
            pltpu.VMEM((_SRCS_BUF,), jnp.int32),
            pltpu.VMEM((_LANES,), jnp.int32),
            pltpu.VMEM((_LANES, _H), jnp.float32),
            pltpu.VMEM((_H,), jnp.float32),
            pltpu.VMEM((_H,), jnp.float32),
            pltpu.VMEM((_H,), jnp.float32),
            pltpu.VMEM((_H,), jnp.float32),
            pltpu.SemaphoreType.DMA,
        ],
    )
    def k(xl_hbm, xr_hbm, srcs_hbm, offs_hbm, att_hbm, bias_hbm, out_hbm,
          off_v, srcs_v, idx16, rows_v, xr_v, att_v, bias_v, acc_v, gsem):
        wid = lax.axis_index("s") * _NC + lax.axis_index("c")
        base = wid * _NPW
        lanes = lax.iota(jnp.int32, _LANES)

        pltpu.sync_copy(offs_hbm.at[pl.ds(base, _OFF_STAGE)], off_v)
        pltpu.sync_copy(att_hbm, att_v)
        pltpu.sync_copy(bias_hbm, bias_v)

        estart = off_v[pl.ds(0, _LANES)][0]
        a0 = (estart // 16) * 16
        pltpu.sync_copy(srcs_hbm.at[pl.ds(a0, _SRCS_BUF)], srcs_v)

        def node_body(i, carry):
            d = base + i

            @pl.when(d < n)
            def _():
                ob = off_v[pl.ds(i, _LANES)]
                a = ob[0]
                b = ob[1]
                deg = b - a
                a_loc = a - a0
                nch = (deg + 15) // 16

                pltpu.sync_copy(xr_hbm.at[d], xr_v)
                for hh in range(_HB):
                    acc_v[pl.ds(hh * 16, 16)] = jnp.zeros((16,), jnp.float32)

                def chunk_body(kk, carry):
                    m_v, denom_v = carry
                    p = a_loc + kk * 16
                    idx16[...] = srcs_v[pl.ds(p, 16)]
                    pltpu.async_copy(xl_hbm.at[idx16], rows_v, gsem).wait()

                    def e_body(j, e16):
                        s = jnp.zeros((16,), jnp.float32)
                        for hh in range(_HB):
                            v = (rows_v[j, pl.ds(hh * 16, 16)]
                                 + xr_v[pl.ds(hh * 16, 16)])
                            v = jnp.maximum(v, 0.2 * v)
                            s = s + v * att_v[pl.ds(hh * 16, 16)]
                        ej = _bfly(s, jnp.add)
                        return jnp.where(lanes == j, ej, e16)

                    e16 = lax.fori_loop(0, _LANES, e_body,
                                        jnp.zeros((16,), jnp.float32))
                    rem = deg - kk * 16
                    e16m = jnp.where(lanes < rem, e16, -1e30)
                    cmax_v = _bfly(e16m, jnp.maximum)
                    m_new = jnp.maximum(m_v, cmax_v)
                    scale_v = jnp.exp(m_v - m_new)
                    w16 = jnp.where(lanes < rem, jnp.exp(e16m - m_new), 0.0)
                    denom_new = denom_v * scale_v + _bfly(w16, jnp.add)

                    for hh in range(_HB):
                        acc_v[pl.ds(hh * 16, 16)] = (
                            acc_v[pl.ds(hh * 16, 16)] * scale_v)

                    def a_body(j, _c):
                        wv = _splat(w16, j)
                        for hh in range(_HB):
                            acc_v[pl.ds(hh * 16, 16)] = (
                                acc_v[pl.ds(hh * 16, 16)]
                                + wv * rows_v[j, pl.ds(hh * 16, 16)])
                        return _c

                    lax.fori_loop(0, _LANES, a_body, 0)
                    return (m_new, denom_new)

                m0 = jnp.full((16,), -1e30, jnp.float32)
                d0 = jnp.zeros((16,), jnp.float32)
                _, denom_v = lax.fori_loop(0, nch, chunk_body, (m0, d0))

                inv_v = 1.0 / (denom_v + 1e-16)
                for hh in range(_HB):
                    ob2 = (acc_v[pl.ds(hh * 16, 16)] * inv_v
                           + bias_v[pl.ds(hh * 16, 16)])
                    acc_v[pl.ds(hh * 16, 16)] = jnp.maximum(ob2, 0.0)
                pltpu.sync_copy(acc_v, out_hbm.at[d])

            return carry

        lax.fori_loop(0, _NPW, node_body, 0)

    return k(xl, xr, srcs_p, offs_p, att, bias)


def _matmul_bias(x, w, b, relu=False):
    m, kdim = x.shape
    n2 = w.shape[1]
    bm = 1000

    def body(xb, wb, bb, ob):
        r = jnp.dot(xb[...], wb[...], preferred_element_type=jnp.float32)
        r = r + bb[...]
        if relu:
            r = jnp.maximum(r, 0.0)
        ob[...] = r

    return pl.pallas_call(
        body,
        grid=(m // bm,),
        in_specs=[
            pl.BlockSpec((bm, kdim), lambda i: (i, 0)),
            pl.BlockSpec((kdim, n2), lambda i: (0, 0)),
            pl.BlockSpec((1, n2), lambda i: (0, 0)),
        ],
        out_specs=pl.BlockSpec((bm, n2), lambda i: (i, 0)),
        out_shape=jax.ShapeDtypeStruct((m, n2), jnp.float32),
    )(x, w, b.reshape(1, -1))


def _mlp_head(h, l1, l2, l3):
    m = h.shape[0]
    bm = 1000
    h1 = l1["W"].shape[1]
    h2 = l2["W"].shape[1]
    c = l3["W"].shape[1]

    def body(hb, w1, b1, w2, b2, w3, b3, logits_ref, probs_ref):
        a = jnp.dot(hb[...], w1[...], preferred_element_type=jnp.float32)
        a = jnp.maximum(a + b1[...], 0.0)
        a = jnp.dot(a, w2[...], preferred_element_type=jnp.float32)
        a = jnp.maximum(a + b2[...], 0.0)
        lg = jnp.dot(a, w3[...], preferred_element_type=jnp.float32) + b3[...]
        logits_ref[...] = lg
        mx = jnp.max(lg, axis=1, keepdims=True)
        ex = jnp.exp(lg - mx)
        probs_ref[...] = ex / jnp.sum(ex, axis=1, keepdims=True)

    return pl.pallas_call(
        body,
        grid=(m // bm,),
        in_specs=[
            pl.BlockSpec((bm, h1), lambda i: (i, 0)),
            pl.BlockSpec((h1, h1), lambda i: (0, 0)),
            pl.BlockSpec((1, h1), lambda i: (0, 0)),
            pl.BlockSpec((h1, h2), lambda i: (0, 0)),
            pl.BlockSpec((1, h2), lambda i: (0, 0)),
            pl.BlockSpec((h2, c), lambda i: (0, 0)),
            pl.BlockSpec((1, c), lambda i: (0, 0)),
        ],
        out_specs=[
            pl.BlockSpec((bm, c), lambda i: (i, 0)),
            pl.BlockSpec((bm, c), lambda i: (i, 0)),
        ],
        out_shape=[
            jax.ShapeDtypeStruct((m, c), jnp.float32),
            jax.ShapeDtypeStruct((m, c), jnp.float32),
        ],
    )(h, l1["W"], l1["b"].reshape(1, -1), l2["W"], l2["b"].reshape(1, -1),
      l3["W"], l3["b"].reshape(1, -1))


def kernel(x, edge_index, params):
    n = x.shape[0]
    e = edge_index.shape[1]
    src, dst = edge_index[0], edge_index[1]

    dst_s, src_s = lax.sort((dst, src), num_keys=1)
    offsets = jnp.searchsorted(
        dst_s, jnp.arange(n + 1, dtype=dst.dtype)).astype(jnp.int32)

    off_len = _NW * _NPW + _OFF_STAGE
    offs_p = jnp.full((off_len,), e, jnp.int32).at[: n + 1].set(offsets)
    srcs_p = jnp.zeros((e + _SRCS_BUF + 16,), jnp.int32).at[:e].set(src_s)

    names = ("gat1", "gat2", "gat3", "gat4")
    din0 = params["gat1"]["Wl"].shape[0]

    def padw(w):
        return jnp.pad(w, ((0, _H - w.shape[0]), (0, 0)))

    stacked = {
        "Wl": jnp.stack([padw(params[k]["Wl"]) for k in names]),
        "bl": jnp.stack([params[k]["bl"] for k in names]),
        "Wr": jnp.stack([padw(params[k]["Wr"]) for k in names]),
        "br": jnp.stack([params[k]["br"] for k in names]),
        "att": jnp.stack([params[k]["att"] for k in names]),
        "bias": jnp.stack([params[k]["bias"] for k in names]),
    }
    h0 = jnp.pad(x, ((0, 0), (0, _H - din0)))

    def layer(h, lp):
        xl = _matmul_bias(h, lp["Wl"], lp["bl"])
        xr = _matmul_bias(h, lp["Wr"], lp["br"])
        return _gat_sc(xl, xr, srcs_p, offs_p, lp["att"], lp["bias"]), None

    h, _ = lax.scan(layer, h0, stacked)

    logits, probs = _mlp_head(h, params["lin1"], params["lin2"], params["lin3"])
    return (logits, probs)


# speedup vs baseline: 2.6543x; 1.3330x over previous
import functools

import jax
import jax.numpy as jnp
from jax import lax
from jax.experimental import pallas as pl
from jax.experimental.pallas import tpu as pltpu
from jax.experimental.pallas import tpu_sc as plsc

_NC = 2
_NS = 16
_NW = _NC * _NS
_LANES = 16

_H = 512
_HB = _H // _LANES
_NPW = 320
_NPAD = _NW * _NPW
_GN = 64
_OFF_STAGE = 336
_SRCS_BUF = 12816

_GATHER_DNUMS = lax.GatherDimensionNumbers(
    offset_dims=(), collapsed_slice_dims=(0,), start_index_map=(0,))


def _shuf(v, idx):
    return lax.gather(v, idx[:, None], _GATHER_DNUMS, slice_sizes=(1,),
                      mode=lax.GatherScatterMode.PROMISE_IN_BOUNDS)


def _bfly(v, op):
    idx = lax.iota(jnp.int32, _LANES)
    for sh in (8, 4, 2, 1):
        v = op(v, _shuf(v, idx ^ sh))
    return v


def _splat(v, j):
    return _shuf(v, jnp.full((_LANES,), j, jnp.int32))


def _gat_sc(xl, xr, srcs_p, offs_p, att, bias):
    mesh = plsc.VectorSubcoreMesh(core_axis_name="c", subcore_axis_name="s")

    @functools.partial(
        pl.kernel,
        mesh=mesh,
        out_type=jax.ShapeDtypeStruct((_NPAD, _H), jnp.float32),
        scratch_types=[
            pltpu.VMEM((_OFF_STAGE,), jnp.int32),
            pltpu.VMEM((_SRCS_BUF,), jnp.int32),
            pltpu.VMEM((_LANES,), jnp.int32),
            pltpu.VMEM((_LANES,), jnp.int32),
            pltpu.VMEM((_LANES, _H), jnp.float32),
            pltpu.VMEM((_LANES, _H), jnp.float32),
            pltpu.VMEM((_GN, _H), jnp.float32),
            pltpu.VMEM((_GN, _H), jnp.float32),
            pltpu.VMEM((_H,), jnp.float32),
            pltpu.VMEM((_H,), jnp.float32),
            pltpu.VMEM((_H,), jnp.float32),
            pltpu.VMEM((_LANES,), jnp.float32),
            pltpu.VMEM((_LANES,), jnp.float32),
            pltpu.SemaphoreType.DMA,
            pltpu.SemaphoreType.DMA,
        ],
    )
    def k(xl_hbm, xr_hbm, srcs_hbm, offs_hbm, att_hbm, bias_hbm, out_hbm,
          off_v, srcs_v, idx0, idx1, rows0, rows1, xr_stage, out_stage,
          att_v, bias_v, acc_v, m_ref, d_ref, sem0, sem1):
        wid = lax.axis_index("s") * _NC + lax.axis_index("c")
        base = wid * _NPW
        lanes = lax.iota(jnp.int32, _LANES)

        pltpu.sync_copy(offs_hbm.at[pl.ds(base, _OFF_STAGE)], off_v)
        pltpu.sync_copy(att_hbm, att_v)
        pltpu.sync_copy(bias_hbm, bias_v)

        estart = off_v[pl.ds(0, _LANES)][0]
        a0 = (estart // 16) * 16
        pltpu.sync_copy(srcs_hbm.at[pl.ds(a0, _SRCS_BUF)], srcs_v)

        bufs = ((idx0, rows0, sem0), (idx1, rows1, sem1))

        def issue(bufidx, p):
            idxb, rowsb, semb = bufs[bufidx]
            idxb[...] = srcs_v[pl.ds(p, 16)]
            pltpu.async_copy(xl_hbm.at[idxb], rowsb, semb)

        def wait(bufidx):
            idxb, rowsb, semb = bufs[bufidx]
            pltpu.make_async_copy(xl_hbm.at[idxb], rowsb, semb).wait()

        def compute(bufidx, kk, deg, il):
            _, rowsb, _ = bufs[bufidx]
            rem = deg - kk * 16
            nj = jnp.minimum(rem, 16)

            def e_body(j, e16):
                s = jnp.zeros((16,), jnp.float32)
                for hh in range(_HB):
                    v = (rowsb[j, pl.ds(hh * 16, 16)]
                         + xr_stage[il, pl.ds(hh * 16, 16)])
                    v = jnp.maximum(v, 0.2 * v)
                    s = s + v * att_v[pl.ds(hh * 16, 16)]
                ej = _bfly(s, jnp.add)
                return jnp.where(lanes == j, ej, e16)

            e16 = lax.fori_loop(0, nj, e_body, jnp.zeros((16,), jnp.float32))
            e16m = jnp.where(lanes < rem, e16, -1e30)
            m_v = m_ref[...]
            m_new = jnp.maximum(m_v, _bfly(e16m, jnp.maximum))
            scale_v = jnp.exp(m_v - m_new)
            w16 = jnp.where(lanes < rem, jnp.exp(e16m - m_new), 0.0)
            m_ref[...] = m_new
            d_ref[...] = d_ref[...] * scale_v + _bfly(w16, jnp.add)

            for hh in range(_HB):
                acc_v[pl.ds(hh * 16, 16)] = acc_v[pl.ds(hh * 16, 16)] * scale_v

            def a_body(j, c):
                wv = _splat(w16, j)
                for hh in range(_HB):
                    acc_v[pl.ds(hh * 16, 16)] = (
                        acc_v[pl.ds(hh * 16, 16)]
                        + wv * rowsb[j, pl.ds(hh * 16, 16)])
                return c

            lax.fori_loop(0, nj, a_body, 0)

        def node_body(i, carry):
            ob = off_v[pl.ds(i, _LANES)]
            a = ob[0]
            b = ob[1]
            deg = b - a
            a_loc = a - a0
            nch = (deg + 15) // 16
            il = lax.rem(i, _GN)

            m_ref[...] = jnp.full((16,), -1e30, jnp.float32)
            d_ref[...] = jnp.zeros((16,), jnp.float32)
            for hh in range(_HB):
                acc_v[pl.ds(hh * 16, 16)] = jnp.zeros((16,), jnp.float32)

            @pl.when(nch > 0)
            def _():
                issue(0, a_loc)

            def pair_body(mm, c):
                k0 = 2 * mm

                wait(0)

                @pl.when(k0 + 1 < nch)
                def _():
                    issue(1, a_loc + (k0 + 1) * 16)

                compute(0, k0, deg, il)

                @pl.when(k0 + 1 < nch)
                def _():
                    wait(1)

                    @pl.when(k0 + 2 < nch)
                    def _():
                        issue(0, a_loc + (k0 + 2) * 16)

                    compute(1, k0 + 1, deg, il)

                return c

            lax.fori_loop(0, (nch + 1) // 2, pair_body, 0)

            inv_v = 1.0 / (d_ref[...] + 1e-16)
            for hh in range(_HB):
                o2 = (acc_v[pl.ds(hh * 16, 16)] * inv_v
                      + bias_v[pl.ds(hh * 16, 16)])
                out_stage[il, pl.ds(hh * 16, 16)] = jnp.maximum(o2, 0.0)

            return carry

        def group_body(g, carry):
            dbase = base + g * _GN
            pltpu.sync_copy(xr_hbm.at[pl.ds(dbase, _GN)], xr_stage)

            def node_in_group(i2, c):
                return node_body_g(g, i2, c)

            lax.fori_loop(0, _GN, node_in_group, 0)
            pltpu.sync_copy(out_stage, out_hbm.at[pl.ds(dbase, _GN)])
            return carry

        def node_body_g(g, i2, c):
            return node_body(g * _GN + i2, c)

        lax.fori_loop(0, _NPW // _GN, group_body, 0)

    return k(xl, xr, srcs_p, offs_p, att, bias)


def _matmul_bias(x, w, b, relu=False):
    m, kdim = x.shape
    n2 = w.shape[1]
    bm = 1024

    def body(xb, wb, bb, ob):
        r = jnp.dot(xb[...], wb[...], preferred_element_type=jnp.float32)
        r = r + bb[...]
        if relu:
            r = jnp.maximum(r, 0.0)
        ob[...] = r

    return pl.pallas_call(
        body,
        grid=(m // bm,),
        in_specs=[
            pl.BlockSpec((bm, kdim), lambda i: (i, 0)),
            pl.BlockSpec((kdim, n2), lambda i: (0, 0)),
            pl.BlockSpec((1, n2), lambda i: (0, 0)),
        ],
        out_specs=pl.BlockSpec((bm, n2), lambda i: (i, 0)),
        out_shape=jax.ShapeDtypeStruct((m, n2), jnp.float32),
    )(x, w, b.reshape(1, -1))


def _mlp_head(h, l1, l2, l3):
    m = h.shape[0]
    bm = 1024
    h1 = l1["W"].shape[1]
    h2 = l2["W"].shape[1]
    c = l3["W"].shape[1]

    def body(hb, w1, b1, w2, b2, w3, b3, logits_ref, probs_ref):
        a = jnp.dot(hb[...], w1[...], preferred_element_type=jnp.float32)
        a = jnp.maximum(a + b1[...], 0.0)
        a = jnp.dot(a, w2[...], preferred_element_type=jnp.float32)
        a = jnp.maximum(a + b2[...], 0.0)
        lg = jnp.dot(a, w3[...], preferred_element_type=jnp.float32) + b3[...]
        logits_ref[...] = lg
        mx = jnp.max(lg, axis=1, keepdims=True)
        ex = jnp.exp(lg - mx)
        probs_ref[...] = ex / jnp.sum(ex, axis=1, keepdims=True)

    return pl.pallas_call(
        body,
        grid=(m // bm,),
        in_specs=[
            pl.BlockSpec((bm, h1), lambda i: (i, 0)),
            pl.BlockSpec((h1, h1), lambda i: (0, 0)),
            pl.BlockSpec((1, h1), lambda i: (0, 0)),
            pl.BlockSpec((h1, h2), lambda i: (0, 0)),
            pl.BlockSpec((1, h2), lambda i: (0, 0)),
            pl.BlockSpec((h2, c), lambda i: (0, 0)),
            pl.BlockSpec((1, c), lambda i: (0, 0)),
        ],
        out_specs=[
            pl.BlockSpec((bm, c), lambda i: (i, 0)),
            pl.BlockSpec((bm, c), lambda i: (i, 0)),
        ],
        out_shape=[
            jax.ShapeDtypeStruct((m, c), jnp.float32),
            jax.ShapeDtypeStruct((m, c), jnp.float32),
        ],
    )(h, l1["W"], l1["b"].reshape(1, -1), l2["W"], l2["b"].reshape(1, -1),
      l3["W"], l3["b"].reshape(1, -1))


def kernel(x, edge_index, params):
    n = x.shape[0]
    e = edge_index.shape[1]
    src, dst = edge_index[0], edge_index[1]

    dst_s, src_s = lax.sort((dst, src), num_keys=1)
    offsets = jnp.searchsorted(
        dst_s, jnp.arange(n + 1, dtype=dst.dtype)).astype(jnp.int32)

    off_len = _NPAD + _OFF_STAGE
    offs_p = jnp.full((off_len,), e, jnp.int32).at[: n + 1].set(offsets)
    srcs_p = jnp.zeros((e + _SRCS_BUF + 16,), jnp.int32).at[:e].set(src_s)

    names = ("gat1", "gat2", "gat3", "gat4")
    din0 = params["gat1"]["Wl"].shape[0]

    def padw(w):
        return jnp.pad(w, ((0, _H - w.shape[0]), (0, 0)))

    stacked = {
        "Wl": jnp.stack([padw(params[k]["Wl"]) for k in names]),
        "bl": jnp.stack([params[k]["bl"] for k in names]),
        "Wr": jnp.stack([padw(params[k]["Wr"]) for k in names]),
        "br": jnp.stack([params[k]["br"] for k in names]),
        "att": jnp.stack([params[k]["att"] for k in names]),
        "bias": jnp.stack([params[k]["bias"] for k in names]),
    }
    h0 = jnp.pad(x, ((0, _NPAD - n), (0, _H - din0)))

    def layer(h, lp):
        xl = _matmul_bias(h, lp["Wl"], lp["bl"])
        xr = _matmul_bias(h, lp["Wr"], lp["br"])
        return _gat_sc(xl, xr, srcs_p, offs_p, lp["att"], lp["bias"]), None

    h, _ = lax.scan(layer, h0, stacked)

    logits, probs = _mlp_head(h, params["lin1"], params["lin2"], params["lin3"])
    return (logits[:n], probs[:n])


# speedup vs baseline: 5.0008x; 1.8840x over previous
import functools

import jax
import jax.numpy as jnp
from jax import lax
from jax.experimental import pallas as pl
from jax.experimental.pallas import tpu as pltpu
from jax.experimental.pallas import tpu_sc as plsc

_NC = 2
_NS = 16
_NW = _NC * _NS
_LANES = 16

_H = 512
_HB = _H // _LANES
_NPW = 320
_NPAD = _NW * _NPW
_GN = 64
_OFF_STAGE = 336
_SRCS_BUF = 12816

_GATHER_DNUMS = lax.GatherDimensionNumbers(
    offset_dims=(), collapsed_slice_dims=(0,), start_index_map=(0,))


def _shuf(v, idx):
    return lax.gather(v, idx[:, None], _GATHER_DNUMS, slice_sizes=(1,),
                      mode=lax.GatherScatterMode.PROMISE_IN_BOUNDS)


def _bfly(v, op):
    idx = lax.iota(jnp.int32, _LANES)
    for sh in (8, 4, 2, 1):
        v = op(v, _shuf(v, idx ^ sh))
    return v


def _splat(v, j):
    return _shuf(v, jnp.full((_LANES,), j, jnp.int32))


def _gat_sc(xl, xr, srcs_p, offs_p, att, bias):
    mesh = plsc.VectorSubcoreMesh(core_axis_name="c", subcore_axis_name="s")

    @functools.partial(
        pl.kernel,
        mesh=mesh,
        out_type=jax.ShapeDtypeStruct((_NPAD, _H), jnp.float32),
        scratch_types=[
            pltpu.VMEM((_OFF_STAGE,), jnp.int32),
            pltpu.VMEM((_SRCS_BUF,), jnp.int32),
            pltpu.VMEM((_LANES,), jnp.int32),
            pltpu.VMEM((_LANES,), jnp.int32),
            pltpu.VMEM((_LANES, _H), jnp.float32),
            pltpu.VMEM((_LANES, _H), jnp.float32),
            pltpu.VMEM((_GN, _H), jnp.float32),
            pltpu.VMEM((_GN, _H), jnp.float32),
            pltpu.VMEM((_H,), jnp.float32),
            pltpu.VMEM((_H,), jnp.float32),
            pltpu.VMEM((_H,), jnp.float32),
            pltpu.VMEM((_LANES,), jnp.float32),
            pltpu.VMEM((_LANES,), jnp.float32),
            pltpu.SemaphoreType.DMA,
            pltpu.SemaphoreType.DMA,
        ],
    )
    def k(xl_hbm, xr_hbm, srcs_hbm, offs_hbm, att_hbm, bias_hbm, out_hbm,
          off_v, srcs_v, idx0, idx1, rows0, rows1, xr_stage, out_stage,
          att_v, bias_v, acc_v, m_ref, d_ref, sem0, sem1):
        wid = lax.axis_index("s") * _NC + lax.axis_index("c")
        base = wid * _NPW
        lanes = lax.iota(jnp.int32, _LANES)

        pltpu.sync_copy(offs_hbm.at[pl.ds(base, _OFF_STAGE)], off_v)
        pltpu.sync_copy(att_hbm, att_v)
        pltpu.sync_copy(bias_hbm, bias_v)

        estart = off_v[pl.ds(0, _LANES)][0]
        a0 = (estart // 16) * 16
        pltpu.sync_copy(srcs_hbm.at[pl.ds(a0, _SRCS_BUF)], srcs_v)

        bufs = ((idx0, rows0, sem0), (idx1, rows1, sem1))

        def issue(bufidx, p):
            idxb, rowsb, semb = bufs[bufidx]
            idxb[...] = srcs_v[pl.ds(p, 16)]
            pltpu.async_copy(xl_hbm.at[idxb], rowsb, semb)

        def wait(bufidx):
            idxb, rowsb, semb = bufs[bufidx]
            pltpu.make_async_copy(xl_hbm.at[idxb], rowsb, semb).wait()

        def compute(bufidx, kk, deg, il):
            _, rowsb, _ = bufs[bufidx]
            rem = deg - kk * 16

            def e_body(hh, s_list):
                xr_b = xr_stage[il, pl.ds(hh * 16, 16)]
                att_b = att_v[pl.ds(hh * 16, 16)]
                out = []
                for j in range(_LANES):
                    v = rowsb[j, pl.ds(hh * 16, 16)] + xr_b
                    v = jnp.maximum(v, 0.2 * v)
                    out.append(s_list[j] + v * att_b)
                return tuple(out)

            s_list = lax.fori_loop(
                0, _HB, e_body,
                tuple(jnp.zeros((16,), jnp.float32) for _ in range(_LANES)))
            e16 = jnp.zeros((16,), jnp.float32)
            for j in range(_LANES):
                e16 = jnp.where(lanes == j, _bfly(s_list[j], jnp.add), e16)

            e16m = jnp.where(lanes < rem, e16, -1e30)
            m_v = m_ref[...]
            m_new = jnp.maximum(m_v, _bfly(e16m, jnp.maximum))
            scale_v = jnp.exp(m_v - m_new)
            w16 = jnp.where(lanes < rem, jnp.exp(e16m - m_new), 0.0)
            m_ref[...] = m_new
            d_ref[...] = d_ref[...] * scale_v + _bfly(w16, jnp.add)

            wv_list = tuple(_splat(w16, j) for j in range(_LANES))

            def a_body(hh, c):
                accb = acc_v[pl.ds(hh * 16, 16)] * scale_v
                for j in range(_LANES):
                    accb = accb + wv_list[j] * rowsb[j, pl.ds(hh * 16, 16)]
                acc_v[pl.ds(hh * 16, 16)] = accb
                return c

            lax.fori_loop(0, _HB, a_body, 0)

        def node_body(i, carry):
            ob = off_v[pl.ds(i, _LANES)]
            a = ob[0]
            b = ob[1]
            deg = b - a
            a_loc = a - a0
            nch = (deg + 15) // 16
            il = lax.rem(i, _GN)

            m_ref[...] = jnp.full((16,), -1e30, jnp.float32)
            d_ref[...] = jnp.zeros((16,), jnp.float32)
            for hh in range(_HB):
                acc_v[pl.ds(hh * 16, 16)] = jnp.zeros((16,), jnp.float32)

            @pl.when(nch > 0)
            def _():
                issue(0, a_loc)

            def pair_body(mm, c):
                k0 = 2 * mm

                wait(0)

                @pl.when(k0 + 1 < nch)
                def _():
                    issue(1, a_loc + (k0 + 1) * 16)

                compute(0, k0, deg, il)

                @pl.when(k0 + 1 < nch)
                def _():
                    wait(1)

                    @pl.when(k0 + 2 < nch)
                    def _():
                        issue(0, a_loc + (k0 + 2) * 16)

                    compute(1, k0 + 1, deg, il)

                return c

            lax.fori_loop(0, (nch + 1) // 2, pair_body, 0)

            inv_v = 1.0 / (d_ref[...] + 1e-16)
            for hh in range(_HB):
                o2 = (acc_v[pl.ds(hh * 16, 16)] * inv_v
                      + bias_v[pl.ds(hh * 16, 16)])
                out_stage[il, pl.ds(hh * 16, 16)] = jnp.maximum(o2, 0.0)

            return carry

        def group_body(g, carry):
            dbase = base + g * _GN
            pltpu.sync_copy(xr_hbm.at[pl.ds(dbase, _GN)], xr_stage)

            def node_in_group(i2, c):
                return node_body_g(g, i2, c)

            lax.fori_loop(0, _GN, node_in_group, 0)
            pltpu.sync_copy(out_stage, out_hbm.at[pl.ds(dbase, _GN)])
            return carry

        def node_body_g(g, i2, c):
            return node_body(g * _GN + i2, c)

        lax.fori_loop(0, _NPW // _GN, group_body, 0)

    return k(xl, xr, srcs_p, offs_p, att, bias)


def _matmul_bias(x, w, b, relu=False):
    m, kdim = x.shape
    n2 = w.shape[1]
    bm = 1024

    def body(xb, wb, bb, ob):
        r = jnp.dot(xb[...], wb[...], preferred_element_type=jnp.float32)
        r = r + bb[...]
        if relu:
            r = jnp.maximum(r, 0.0)
        ob[...] = r

    return pl.pallas_call(
        body,
        grid=(m // bm,),
        in_specs=[
            pl.BlockSpec((bm, kdim), lambda i: (i, 0)),
            pl.BlockSpec((kdim, n2), lambda i: (0, 0)),
            pl.BlockSpec((1, n2), lambda i: (0, 0)),
        ],
        out_specs=pl.BlockSpec((bm, n2), lambda i: (i, 0)),
        out_shape=jax.ShapeDtypeStruct((m, n2), jnp.float32),
    )(x, w, b.reshape(1, -1))


def _mlp_head(h, l1, l2, l3):
    m = h.shape[0]
    bm = 1024
    h1 = l1["W"].shape[1]
    h2 = l2["W"].shape[1]
    c = l3["W"].shape[1]

    def body(hb, w1, b1, w2, b2, w3, b3, logits_ref, probs_ref):
        a = jnp.dot(hb[...], w1[...], preferred_element_type=jnp.float32)
        a = jnp.maximum(a + b1[...], 0.0)
        a = jnp.dot(a, w2[...], preferred_element_type=jnp.float32)
        a = jnp.maximum(a + b2[...], 0.0)
        lg = jnp.dot(a, w3[...], preferred_element_type=jnp.float32) + b3[...]
        logits_ref[...] = lg
        mx = jnp.max(lg, axis=1, keepdims=True)
        ex = jnp.exp(lg - mx)
        probs_ref[...] = ex / jnp.sum(ex, axis=1, keepdims=True)

    return pl.pallas_call(
        body,
        grid=(m // bm,),
        in_specs=[
            pl.BlockSpec((bm, h1), lambda i: (i, 0)),
            pl.BlockSpec((h1, h1), lambda i: (0, 0)),
            pl.BlockSpec((1, h1), lambda i: (0, 0)),
            pl.BlockSpec((h1, h2), lambda i: (0, 0)),
            pl.BlockSpec((1, h2), lambda i: (0, 0)),
            pl.BlockSpec((h2, c), lambda i: (0, 0)),
            pl.BlockSpec((1, c), lambda i: (0, 0)),
        ],
        out_specs=[
            pl.BlockSpec((bm, c), lambda i: (i, 0)),
            pl.BlockSpec((bm, c), lambda i: (i, 0)),
        ],
        out_shape=[
            jax.ShapeDtypeStruct((m, c), jnp.float32),
            jax.ShapeDtypeStruct((m, c), jnp.float32),
        ],
    )(h, l1["W"], l1["b"].reshape(1, -1), l2["W"], l2["b"].reshape(1, -1),
      l3["W"], l3["b"].reshape(1, -1))


def kernel(x, edge_index, params):
    n = x.shape[0]
    e = edge_index.shape[1]
    src, dst = edge_index[0], edge_index[1]

    dst_s, src_s = lax.sort((dst, src), num_keys=1)
    offsets = jnp.searchsorted(
        dst_s, jnp.arange(n + 1, dtype=dst.dtype)).astype(jnp.int32)

    off_len = _NPAD + _OFF_STAGE
    offs_p = jnp.full((off_len,), e, jnp.int32).at[: n + 1].set(offsets)
    srcs_p = jnp.zeros((e + _SRCS_BUF + 16,), jnp.int32).at[:e].set(src_s)

    names = ("gat1", "gat2", "gat3", "gat4")
    din0 = params["gat1"]["Wl"].shape[0]

    def padw(w):
        return jnp.pad(w, ((0, _H - w.shape[0]), (0, 0)))

    stacked = {
        "Wl": jnp.stack([padw(params[k]["Wl"]) for k in names]),
        "bl": jnp.stack([params[k]["bl"] for k in names]),
        "Wr": jnp.stack([padw(params[k]["Wr"]) for k in names]),
        "br": jnp.stack([params[k]["br"] for k in names]),
        "att": jnp.stack([params[k]["att"] for k in names]),
        "bias": jnp.stack([params[k]["bias"] for k in names]),
    }
    h0 = jnp.pad(x, ((0, _NPAD - n), (0, _H - din0)))

    def layer(h, lp):
        xl = _matmul_bias(h, lp["Wl"], lp["bl"])
        xr = _matmul_bias(h, lp["Wr"], lp["br"])
        return _gat_sc(xl, xr, srcs_p, offs_p, lp["att"], lp["bias"]), None

    h, _ = lax.scan(layer, h0, stacked)

    logits, probs = _mlp_head(h, params["lin1"], params["lin2"], params["lin3"])
    return (logits[:n], probs[:n])
